# Initial kernel scaffold; baseline (speedup 1.0000x reference)
import jax
import jax.numpy as jnp
from jax.experimental import pallas as pl


def kernel(conv1_w, conv1_b, conv2_w, conv2_b, fc1_w_t, fc1_b, fc2_w_t, fc2_b, fc3_w_t, fc3_b, x):
    raise NotImplementedError("write your pallas kernel here")


# speedup vs baseline: 5.1125x; 5.1125x over previous
import jax
import jax.numpy as jnp
from jax import lax
from jax.experimental import pallas as pl
from jax.experimental.pallas import tpu as pltpu


def _conv1_kernel(w_ref, b_ref, x_ref, o_ref):
    n_co = 6
    phases = ((0, 0), (0, 1), (1, 0), (1, 1))

    def strip_body(s, _):
        p0 = s * 8

        def tap_body(cidy, accs):
            ci = cidy // 5
            dy = cidy % 5
            accs = list(accs)
            for dx in range(5):
                t = cidy * 5 + dx
                sls = []
                for (ry, rx) in phases:
                    by = (ry + dy) % 2
                    ay = (ry + dy) // 2
                    bx = (rx + dx) % 2
                    ax = (rx + dx) // 2
                    sls.append(x_ref[0, ci, by, bx,
                                     pl.ds(p0 + ay, 8), ax:ax + 120])
                for co in range(n_co):
                    w = w_ref[co, t]
                    for ph in range(4):
                        i = ph * n_co + co
                        accs[i] = accs[i] + w * sls[ph]
            return tuple(accs)

        zero = jnp.zeros((8, 120), jnp.float32)
        accs = lax.fori_loop(0, 15, tap_body, (zero,) * (4 * n_co))
        for co in range(n_co):
            m = jnp.maximum(jnp.maximum(accs[co], accs[n_co + co]),
                            jnp.maximum(accs[2 * n_co + co],
                                        accs[3 * n_co + co]))
            o_ref[0, co, pl.ds(p0, 8), :] = jnp.maximum(m + b_ref[co, 0], 0.0)
        return 0

    lax.fori_loop(0, 15, strip_body, 0)


def _conv1(x, w, b):
    n = x.shape[0]
    xq = x.reshape(n, 3, 122, 2, 122, 2).transpose(0, 1, 3, 5, 2, 4)
    w2 = w.reshape(6, 75)
    b2 = b.reshape(6, 1)
    return pl.pallas_call(
        _conv1_kernel,
        out_shape=jax.ShapeDtypeStruct((n, 6, 120, 120), jnp.float32),
        grid=(n,),
        in_specs=[
            pl.BlockSpec(memory_space=pltpu.SMEM),
            pl.BlockSpec(memory_space=pltpu.SMEM),
            pl.BlockSpec((1, 3, 2, 2, 122, 122),
                         lambda i: (i, 0, 0, 0, 0, 0)),
        ],
        out_specs=pl.BlockSpec((1, 6, 120, 120), lambda i: (i, 0, 0, 0)),
        compiler_params=pltpu.CompilerParams(
            dimension_semantics=("parallel",)),
    )(w2, b2, xq)


def _conv2_kernel(w_ref, b_ref, x_ref, o_ref):
    n_co = 16

    def strip_body(s, _):
        p0 = s * 8

        def tap_body(cidy, accs):
            ci = cidy // 5
            dy = cidy % 5
            accs = list(accs)
            for dx in range(5):
                t = cidy * 5 + dx
                bx0 = dx % 2
                ax0 = dx // 2
                bx1 = (1 + dx) % 2
                ax1 = (1 + dx) // 2
                sls = []
                for ry in range(2):
                    by = (ry + dy) % 2
                    ay = (ry + dy) // 2
                    h0 = x_ref[0, ci, by, bx0, pl.ds(p0 + ay, 8),
                               ax0:ax0 + 58]
                    h1 = x_ref[0, ci, by, bx1, pl.ds(p0 + ay, 8),
                               ax1:ax1 + 58]
                    sls.append(jnp.concatenate([h0, h1], axis=1))
                for co in range(n_co):
                    w = w_ref[co, t]
                    accs[co] = accs[co] + w * sls[0]
                    accs[n_co + co] = accs[n_co + co] + w * sls[1]
            return tuple(accs)

        zero = jnp.zeros((8, 116), jnp.float32)
        accs = lax.fori_loop(0, 30, tap_body, (zero,) * (2 * n_co))
        for co in range(n_co):
            m = jnp.maximum(accs[co], accs[n_co + co])
            m = jnp.maximum(m[:, :58], m[:, 58:])
            o_ref[0, co, pl.ds(p0, 8), :] = jnp.maximum(m + b_ref[co, 0], 0.0)
        return 0

    lax.fori_loop(0, 8, strip_body, 0)


def _conv2(x1, w, b):
    n = x1.shape[0]
    xq = x1.reshape(n, 6, 60, 2, 60, 2).transpose(0, 1, 3, 5, 2, 4)
    xq = jnp.pad(xq, ((0, 0), (0, 0), (0, 0), (0, 0), (0, 12), (0, 0)))
    w2 = w.reshape(16, 150)
    b2 = b.reshape(16, 1)
    out = pl.pallas_call(
        _conv2_kernel,
        out_shape=jax.ShapeDtypeStruct((n, 16, 64, 58), jnp.float32),
        grid=(n,),
        in_specs=[
            pl.BlockSpec(memory_space=pltpu.SMEM),
            pl.BlockSpec(memory_space=pltpu.SMEM),
            pl.BlockSpec((1, 6, 2, 2, 72, 60),
                         lambda i: (i, 0, 0, 0, 0, 0)),
        ],
        out_specs=pl.BlockSpec((1, 16, 64, 58), lambda i: (i, 0, 0, 0)),
        compiler_params=pltpu.CompilerParams(
            dimension_semantics=("parallel",)),
    )(w2, b2, xq)
    return out[:, :, :58, :]


FC1_TK = 8192


def _fc1_kernel(x_ref, w_ref, b_ref, o_ref, acc_ref):
    k = pl.program_id(1)

    @pl.when(k == 0)
    def _():
        acc_ref[...] = jnp.zeros_like(acc_ref)

    acc_ref[...] += jnp.dot(x_ref[...], w_ref[...],
                            preferred_element_type=jnp.float32)

    @pl.when(k == pl.num_programs(1) - 1)
    def _():
        o_ref[...] = jnp.maximum(acc_ref[...] + b_ref[...], 0.0)


def _fc23_kernel(h_ref, w2_ref, b2_ref, w3_ref, b3_ref, o_ref):
    h2 = jnp.maximum(
        jnp.dot(h_ref[...], w2_ref[...], preferred_element_type=jnp.float32)
        + b2_ref[...], 0.0)
    o_ref[...] = (jnp.dot(h2, w3_ref[...], preferred_element_type=jnp.float32)
                  + b3_ref[...])


def _fc_head(x2, w1, b1, w2, b2, w3, b3):
    m = x2.shape[0]
    kp = w1.shape[0]
    xp = jnp.pad(x2, ((0, 0), (0, kp - x2.shape[1])))
    nk = kp // FC1_TK
    h = pl.pallas_call(
        _fc1_kernel,
        out_shape=jax.ShapeDtypeStruct((m, 256), jnp.float32),
        grid=(2, nk),
        in_specs=[
            pl.BlockSpec((m, FC1_TK), lambda nh, k: (0, k)),
            pl.BlockSpec((FC1_TK, 128), lambda nh, k: (k, nh)),
            pl.BlockSpec((1, 128), lambda nh, k: (0, nh)),
        ],
        out_specs=pl.BlockSpec((m, 128), lambda nh, k: (0, nh)),
        scratch_shapes=[pltpu.VMEM((m, 128), jnp.float32)],
        compiler_params=pltpu.CompilerParams(
            dimension_semantics=("parallel", "arbitrary")),
    )(xp, w1, b1.reshape(1, 256))

    b2p = jnp.pad(b2, (0, 128 - b2.shape[0])).reshape(1, 128)
    b3p = jnp.pad(b3, (0, 128 - b3.shape[0])).reshape(1, 128)
    out = pl.pallas_call(
        _fc23_kernel,
        out_shape=jax.ShapeDtypeStruct((m, 128), jnp.float32),
    )(h, w2, b2p, w3, b3p)
    return out[:, :10]


def kernel(conv1_w, conv1_b, conv2_w, conv2_b, fc1_w_t, fc1_b,
           fc2_w_t, fc2_b, fc3_w_t, fc3_b, x):
    x1 = _conv1(x, conv1_w, conv1_b)
    x2 = _conv2(x1, conv2_w, conv2_b)
    n = x2.shape[0]
    flat = x2.reshape(n, 16 * 58 * 58)
    return _fc_head(flat, fc1_w_t, fc1_b, fc2_w_t, fc2_b, fc3_w_t, fc3_b)


# speedup vs baseline: 9.3211x; 1.8232x over previous
import jax
import jax.numpy as jnp
from jax import lax
from jax.experimental import pallas as pl
from jax.experimental.pallas import tpu as pltpu


def _conv1_kernel(w_ref, b_ref, x_ref, o_ref):
    n_co = 6
    phases = ((0, 0), (0, 1), (1, 0), (1, 1))

    def strip_body(s, _):
        p0 = s * 8

        def tap_body(cidy, accs):
            ci = cidy // 5
            dy = cidy % 5
            accs = list(accs)
            for dx in range(5):
                t = cidy * 5 + dx
                sls = []
                for (ry, rx) in phases:
                    by = (ry + dy) % 2
                    ay = (ry + dy) // 2
                    sls.append(x_ref[0, ci, by, rx + dx,
                                     pl.ds(p0 + ay, 8), :])
                for co in range(n_co):
                    w = w_ref[co, t]
                    for ph in range(4):
                        i = ph * n_co + co
                        accs[i] = accs[i] + w * sls[ph]
            return tuple(accs)

        zero = jnp.zeros((8, 120), jnp.float32)
        accs = lax.fori_loop(0, 15, tap_body, (zero,) * (4 * n_co))
        for co in range(n_co):
            m = jnp.maximum(jnp.maximum(accs[co], accs[n_co + co]),
                            jnp.maximum(accs[2 * n_co + co],
                                        accs[3 * n_co + co]))
            o_ref[0, co, pl.ds(p0, 8), :] = jnp.maximum(m + b_ref[co, 0], 0.0)
        return 0

    lax.fori_loop(0, 15, strip_body, 0)


def _conv1(x, w, b):
    n = x.shape[0]
    xq = x.reshape(n, 3, 122, 2, 122, 2).transpose(0, 1, 3, 5, 2, 4)
    xsh = jnp.stack([xq[:, :, :, s % 2, :, s // 2:s // 2 + 120]
                     for s in range(6)], axis=3)
    w2 = w.reshape(6, 75)
    b2 = b.reshape(6, 1)
    return pl.pallas_call(
        _conv1_kernel,
        out_shape=jax.ShapeDtypeStruct((n, 6, 120, 120), jnp.float32),
        grid=(n,),
        in_specs=[
            pl.BlockSpec(memory_space=pltpu.SMEM),
            pl.BlockSpec(memory_space=pltpu.SMEM),
            pl.BlockSpec((1, 3, 2, 6, 122, 120),
                         lambda i: (i, 0, 0, 0, 0, 0)),
        ],
        out_specs=pl.BlockSpec((1, 6, 120, 120), lambda i: (i, 0, 0, 0)),
        compiler_params=pltpu.CompilerParams(
            dimension_semantics=("parallel",)),
    )(w2, b2, xsh)


def _conv2_kernel(w_ref, b_ref, x_ref, o_ref):
    n_co = 16

    def strip_body(s, _):
        p0 = s * 8

        def tap_body(cidy, accs):
            ci = cidy // 5
            dy = cidy % 5
            accs = list(accs)
            for dx in range(5):
                t = cidy * 5 + dx
                sls = []
                for ry in range(2):
                    by = (ry + dy) % 2
                    ay = (ry + dy) // 2
                    sls.append(x_ref[0, ci, by, dx, pl.ds(p0 + ay, 8), :])
                for co in range(n_co):
                    w = w_ref[co, t]
                    accs[co] = accs[co] + w * sls[0]
                    accs[n_co + co] = accs[n_co + co] + w * sls[1]
            return tuple(accs)

        zero = jnp.zeros((8, 116), jnp.float32)
        accs = lax.fori_loop(0, 30, tap_body, (zero,) * (2 * n_co))
        for co in range(n_co):
            m = jnp.maximum(accs[co], accs[n_co + co])
            m = jnp.maximum(m[:, :58], m[:, 58:])
            o_ref[0, co, pl.ds(p0, 8), :] = jnp.maximum(m + b_ref[co, 0], 0.0)
        return 0

    lax.fori_loop(0, 8, strip_body, 0)


def _conv2(x1, w, b):
    n = x1.shape[0]
    xq = x1.reshape(n, 6, 60, 2, 60, 2).transpose(0, 1, 3, 5, 2, 4)
    xq = jnp.pad(xq, ((0, 0), (0, 0), (0, 0), (0, 0), (0, 12), (0, 0)))
    xp = jnp.stack(
        [jnp.concatenate(
            [xq[:, :, :, (rx + dx) % 2, :,
                (rx + dx) // 2:(rx + dx) // 2 + 58] for rx in range(2)],
            axis=-1)
         for dx in range(5)], axis=3)
    w2 = w.reshape(16, 150)
    b2 = b.reshape(16, 1)
    out = pl.pallas_call(
        _conv2_kernel,
        out_shape=jax.ShapeDtypeStruct((n, 16, 64, 58), jnp.float32),
        grid=(n,),
        in_specs=[
            pl.BlockSpec(memory_space=pltpu.SMEM),
            pl.BlockSpec(memory_space=pltpu.SMEM),
            pl.BlockSpec((1, 6, 2, 5, 72, 116),
                         lambda i: (i, 0, 0, 0, 0, 0)),
        ],
        out_specs=pl.BlockSpec((1, 16, 64, 58), lambda i: (i, 0, 0, 0)),
        compiler_params=pltpu.CompilerParams(
            dimension_semantics=("parallel",)),
    )(w2, b2, xp)
    return out[:, :, :58, :]


FC1_TK = 8192


def _fc1_kernel(x_ref, w_ref, b_ref, o_ref, acc_ref):
    k = pl.program_id(1)

    @pl.when(k == 0)
    def _():
        acc_ref[...] = jnp.zeros_like(acc_ref)

    acc_ref[...] += jnp.dot(x_ref[...], w_ref[...],
                            preferred_element_type=jnp.float32)

    @pl.when(k == pl.num_programs(1) - 1)
    def _():
        o_ref[...] = jnp.maximum(acc_ref[...] + b_ref[...], 0.0)


def _fc23_kernel(h_ref, w2_ref, b2_ref, w3_ref, b3_ref, o_ref):
    h2 = jnp.maximum(
        jnp.dot(h_ref[...], w2_ref[...], preferred_element_type=jnp.float32)
        + b2_ref[...], 0.0)
    o_ref[...] = (jnp.dot(h2, w3_ref[...], preferred_element_type=jnp.float32)
                  + b3_ref[...])


def _fc_head(x2, w1, b1, w2, b2, w3, b3):
    m = x2.shape[0]
    kp = w1.shape[0]
    xp = jnp.pad(x2, ((0, 0), (0, kp - x2.shape[1])))
    nk = kp // FC1_TK
    h = pl.pallas_call(
        _fc1_kernel,
        out_shape=jax.ShapeDtypeStruct((m, 256), jnp.float32),
        grid=(2, nk),
        in_specs=[
            pl.BlockSpec((m, FC1_TK), lambda nh, k: (0, k)),
            pl.BlockSpec((FC1_TK, 128), lambda nh, k: (k, nh)),
            pl.BlockSpec((1, 128), lambda nh, k: (0, nh)),
        ],
        out_specs=pl.BlockSpec((m, 128), lambda nh, k: (0, nh)),
        scratch_shapes=[pltpu.VMEM((m, 128), jnp.float32)],
        compiler_params=pltpu.CompilerParams(
            dimension_semantics=("parallel", "arbitrary")),
    )(xp, w1, b1.reshape(1, 256))

    b2p = jnp.pad(b2, (0, 128 - b2.shape[0])).reshape(1, 128)
    b3p = jnp.pad(b3, (0, 128 - b3.shape[0])).reshape(1, 128)
    out = pl.pallas_call(
        _fc23_kernel,
        out_shape=jax.ShapeDtypeStruct((m, 128), jnp.float32),
    )(h, w2, b2p, w3, b3p)
    return out[:, :10]


def kernel(conv1_w, conv1_b, conv2_w, conv2_b, fc1_w_t, fc1_b,
           fc2_w_t, fc2_b, fc3_w_t, fc3_b, x):
    x1 = _conv1(x, conv1_w, conv1_b)
    x2 = _conv2(x1, conv2_w, conv2_b)
    n = x2.shape[0]
    flat = x2.reshape(n, 16 * 58 * 58)
    return _fc_head(flat, fc1_w_t, fc1_b, fc2_w_t, fc2_b, fc3_w_t, fc3_b)


# speedup vs baseline: 14.0800x; 1.5105x over previous
import jax
import jax.numpy as jnp
from jax import lax
from jax.experimental import pallas as pl
from jax.experimental.pallas import tpu as pltpu


def _conv1_kernel(w_ref, b_ref, x_ref, o_ref, acc_ref):
    n_co = 6
    phases = ((0, 0), (0, 1), (1, 0), (1, 1))

    acc_ref[...] = jnp.zeros_like(acc_ref)

    def tap_body(cidy, carry):
        ci = cidy // 5
        dy = cidy % 5
        ws = [[w_ref[co, cidy * 5 + dx] for dx in range(5)]
              for co in range(n_co)]
        bys = []
        ays = []
        for ry in range(2):
            bys.append((ry + dy) % 2)
            ays.append((ry + dy) // 2)
        for st in range(5):
            p0 = 24 * st
            sl = [[x_ref[0, ci, bys[ry], s, pl.ds(p0 + ays[ry], 24), :]
                   for s in range(6)] for ry in range(2)]
            for ph, (ry, rx) in enumerate(phases):
                for co in range(n_co):
                    a = acc_ref[ph * n_co + co, pl.ds(p0, 24), :]
                    for dx in range(5):
                        a = a + ws[co][dx] * sl[ry][rx + dx]
                    acc_ref[ph * n_co + co, pl.ds(p0, 24), :] = a
        return carry

    lax.fori_loop(0, 15, tap_body, 0)

    for co in range(n_co):
        m = jnp.maximum(jnp.maximum(acc_ref[co], acc_ref[n_co + co]),
                        jnp.maximum(acc_ref[2 * n_co + co],
                                    acc_ref[3 * n_co + co]))
        o_ref[0, co] = jnp.maximum(m + b_ref[co, 0], 0.0)


def _conv1(x, w, b):
    n = x.shape[0]
    xq = x.reshape(n, 3, 122, 2, 122, 2).transpose(0, 1, 3, 5, 2, 4)
    xsh = jnp.stack([xq[:, :, :, s % 2, :, s // 2:s // 2 + 120]
                     for s in range(6)], axis=3)
    w2 = w.reshape(6, 75)
    b2 = b.reshape(6, 1)
    return pl.pallas_call(
        _conv1_kernel,
        out_shape=jax.ShapeDtypeStruct((n, 6, 120, 120), jnp.float32),
        grid=(n,),
        in_specs=[
            pl.BlockSpec(memory_space=pltpu.SMEM),
            pl.BlockSpec(memory_space=pltpu.SMEM),
            pl.BlockSpec((1, 3, 2, 6, 122, 120),
                         lambda i: (i, 0, 0, 0, 0, 0)),
        ],
        out_specs=pl.BlockSpec((1, 6, 120, 120), lambda i: (i, 0, 0, 0)),
        scratch_shapes=[pltpu.VMEM((24, 120, 120), jnp.float32)],
        compiler_params=pltpu.CompilerParams(
            dimension_semantics=("parallel",)),
    )(w2, b2, xsh)


def _conv2_kernel(w_ref, b_ref, x_ref, o_ref, acc_ref):
    n_co = 16

    acc_ref[...] = jnp.zeros_like(acc_ref)

    def tap_body(cidy, carry):
        ci = cidy // 5
        dy = cidy % 5
        ws = [[w_ref[co, cidy * 5 + dx] for dx in range(5)]
              for co in range(n_co)]
        for st in range(4):
            p0 = 16 * st
            sl = []
            for ry in range(2):
                by = (ry + dy) % 2
                ay = (ry + dy) // 2
                sl.append([x_ref[0, ci, by, dx, pl.ds(p0 + ay, 16), :]
                           for dx in range(5)])
            for ry in range(2):
                for co in range(n_co):
                    a = acc_ref[ry * n_co + co, pl.ds(p0, 16), :]
                    for dx in range(5):
                        a = a + ws[co][dx] * sl[ry][dx]
                    acc_ref[ry * n_co + co, pl.ds(p0, 16), :] = a
        return carry

    lax.fori_loop(0, 30, tap_body, 0)

    for co in range(n_co):
        m = jnp.maximum(acc_ref[co], acc_ref[n_co + co])
        m = jnp.maximum(m[:, :58], m[:, 58:])
        o_ref[0, co] = jnp.maximum(m + b_ref[co, 0], 0.0)


def _conv2(x1, w, b):
    n = x1.shape[0]
    xq = x1.reshape(n, 6, 60, 2, 60, 2).transpose(0, 1, 3, 5, 2, 4)
    xq = jnp.pad(xq, ((0, 0), (0, 0), (0, 0), (0, 0), (0, 12), (0, 0)))
    xp = jnp.stack(
        [jnp.concatenate(
            [xq[:, :, :, (rx + dx) % 2, :,
                (rx + dx) // 2:(rx + dx) // 2 + 58] for rx in range(2)],
            axis=-1)
         for dx in range(5)], axis=3)
    w2 = w.reshape(16, 150)
    b2 = b.reshape(16, 1)
    out = pl.pallas_call(
        _conv2_kernel,
        out_shape=jax.ShapeDtypeStruct((n, 16, 64, 58), jnp.float32),
        grid=(n,),
        in_specs=[
            pl.BlockSpec(memory_space=pltpu.SMEM),
            pl.BlockSpec(memory_space=pltpu.SMEM),
            pl.BlockSpec((1, 6, 2, 5, 72, 116),
                         lambda i: (i, 0, 0, 0, 0, 0)),
        ],
        out_specs=pl.BlockSpec((1, 16, 64, 58), lambda i: (i, 0, 0, 0)),
        scratch_shapes=[pltpu.VMEM((32, 64, 116), jnp.float32)],
        compiler_params=pltpu.CompilerParams(
            dimension_semantics=("parallel",)),
    )(w2, b2, xp)
    return out[:, :, :58, :]


FC1_TK = 8192


def _fc1_kernel(x_ref, w_ref, b_ref, o_ref, acc_ref):
    k = pl.program_id(1)

    @pl.when(k == 0)
    def _():
        acc_ref[...] = jnp.zeros_like(acc_ref)

    acc_ref[...] += jnp.dot(x_ref[...], w_ref[...],
                            preferred_element_type=jnp.float32)

    @pl.when(k == pl.num_programs(1) - 1)
    def _():
        o_ref[...] = jnp.maximum(acc_ref[...] + b_ref[...], 0.0)


def _fc23_kernel(h_ref, w2_ref, b2_ref, w3_ref, b3_ref, o_ref):
    h2 = jnp.maximum(
        jnp.dot(h_ref[...], w2_ref[...], preferred_element_type=jnp.float32)
        + b2_ref[...], 0.0)
    o_ref[...] = (jnp.dot(h2, w3_ref[...], preferred_element_type=jnp.float32)
                  + b3_ref[...])


def _fc_head(x2, w1, b1, w2, b2, w3, b3):
    m = x2.shape[0]
    kp = w1.shape[0]
    xp = jnp.pad(x2, ((0, 0), (0, kp - x2.shape[1])))
    nk = kp // FC1_TK
    h = pl.pallas_call(
        _fc1_kernel,
        out_shape=jax.ShapeDtypeStruct((m, 256), jnp.float32),
        grid=(2, nk),
        in_specs=[
            pl.BlockSpec((m, FC1_TK), lambda nh, k: (0, k)),
            pl.BlockSpec((FC1_TK, 128), lambda nh, k: (k, nh)),
            pl.BlockSpec((1, 128), lambda nh, k: (0, nh)),
        ],
        out_specs=pl.BlockSpec((m, 128), lambda nh, k: (0, nh)),
        scratch_shapes=[pltpu.VMEM((m, 128), jnp.float32)],
        compiler_params=pltpu.CompilerParams(
            dimension_semantics=("parallel", "arbitrary")),
    )(xp, w1, b1.reshape(1, 256))

    b2p = jnp.pad(b2, (0, 128 - b2.shape[0])).reshape(1, 128)
    b3p = jnp.pad(b3, (0, 128 - b3.shape[0])).reshape(1, 128)
    out = pl.pallas_call(
        _fc23_kernel,
        out_shape=jax.ShapeDtypeStruct((m, 128), jnp.float32),
    )(h, w2, b2p, w3, b3p)
    return out[:, :10]


def kernel(conv1_w, conv1_b, conv2_w, conv2_b, fc1_w_t, fc1_b,
           fc2_w_t, fc2_b, fc3_w_t, fc3_b, x):
    x1 = _conv1(x, conv1_w, conv1_b)
    x2 = _conv2(x1, conv2_w, conv2_b)
    n = x2.shape[0]
    flat = x2.reshape(n, 16 * 58 * 58)
    return _fc_head(flat, fc1_w_t, fc1_b, fc2_w_t, fc2_b, fc3_w_t, fc3_b)


# speedup vs baseline: 16.5499x; 1.1754x over previous
import jax
import jax.numpy as jnp
from jax import lax
from jax.experimental import pallas as pl
from jax.experimental.pallas import tpu as pltpu


def _conv1_kernel(w_ref, b_ref, xq_ref, o_ref, xs_ref, acc_ref):
    n_co = 6
    phases = ((0, 0), (0, 1), (1, 0), (1, 1))

    for ci in range(3):
        for by in range(2):
            for s in range(6):
                xs_ref[ci, by, s] = xq_ref[0, ci, by, s % 2, :,
                                           s // 2:s // 2 + 120]

    acc_ref[...] = jnp.zeros_like(acc_ref)

    def tap_body(cidy, carry):
        ci = cidy // 5
        dy = cidy % 5
        ws = [[w_ref[co, cidy * 5 + dx] for dx in range(5)]
              for co in range(n_co)]
        bys = []
        ays = []
        for ry in range(2):
            bys.append((ry + dy) % 2)
            ays.append((ry + dy) // 2)
        for st in range(5):
            p0 = 24 * st
            sl = [[xs_ref[ci, bys[ry], s, pl.ds(p0 + ays[ry], 24), :]
                   for s in range(6)] for ry in range(2)]
            for ph, (ry, rx) in enumerate(phases):
                for co in range(n_co):
                    a = acc_ref[ph * n_co + co, pl.ds(p0, 24), :]
                    for dx in range(5):
                        a = a + ws[co][dx] * sl[ry][rx + dx]
                    acc_ref[ph * n_co + co, pl.ds(p0, 24), :] = a
        return carry

    lax.fori_loop(0, 15, tap_body, 0)

    for co in range(n_co):
        m = jnp.maximum(jnp.maximum(acc_ref[co], acc_ref[n_co + co]),
                        jnp.maximum(acc_ref[2 * n_co + co],
                                    acc_ref[3 * n_co + co]))
        o_ref[0, co] = jnp.maximum(m + b_ref[co, 0], 0.0)


def _conv1(x, w, b):
    n = x.shape[0]
    xq = x.reshape(n, 3, 122, 2, 122, 2).transpose(0, 1, 3, 5, 2, 4)
    w2 = w.reshape(6, 75)
    b2 = b.reshape(6, 1)
    return pl.pallas_call(
        _conv1_kernel,
        out_shape=jax.ShapeDtypeStruct((n, 6, 120, 120), jnp.float32),
        grid=(n,),
        in_specs=[
            pl.BlockSpec(memory_space=pltpu.SMEM),
            pl.BlockSpec(memory_space=pltpu.SMEM),
            pl.BlockSpec((1, 3, 2, 2, 122, 122),
                         lambda i: (i, 0, 0, 0, 0, 0)),
        ],
        out_specs=pl.BlockSpec((1, 6, 120, 120), lambda i: (i, 0, 0, 0)),
        scratch_shapes=[pltpu.VMEM((3, 2, 6, 122, 120), jnp.float32),
                        pltpu.VMEM((24, 120, 120), jnp.float32)],
        compiler_params=pltpu.CompilerParams(
            dimension_semantics=("parallel",)),
    )(w2, b2, xq)


def _conv2_kernel(w_ref, b_ref, xq_ref, o_ref, xp_ref, acc_ref):
    n_co = 16

    for ci in range(6):
        for by in range(2):
            for dx in range(5):
                xp_ref[ci, by, dx] = jnp.concatenate(
                    [xq_ref[0, ci, by, (rx + dx) % 2, :,
                            (rx + dx) // 2:(rx + dx) // 2 + 58]
                     for rx in range(2)], axis=-1)

    acc_ref[...] = jnp.zeros_like(acc_ref)

    def tap_body(cidy, carry):
        ci = cidy // 5
        dy = cidy % 5
        ws = [[w_ref[co, cidy * 5 + dx] for dx in range(5)]
              for co in range(n_co)]
        for st in range(4):
            p0 = 16 * st
            sl = []
            for ry in range(2):
                by = (ry + dy) % 2
                ay = (ry + dy) // 2
                sl.append([xp_ref[ci, by, dx, pl.ds(p0 + ay, 16), :]
                           for dx in range(5)])
            for ry in range(2):
                for co in range(n_co):
                    a = acc_ref[ry * n_co + co, pl.ds(p0, 16), :]
                    for dx in range(5):
                        a = a + ws[co][dx] * sl[ry][dx]
                    acc_ref[ry * n_co + co, pl.ds(p0, 16), :] = a
        return carry

    lax.fori_loop(0, 30, tap_body, 0)

    for co in range(n_co):
        m = jnp.maximum(acc_ref[co], acc_ref[n_co + co])
        m = jnp.maximum(m[:, :58], m[:, 58:])
        o_ref[0, co] = jnp.maximum(m[:58, :] + b_ref[co, 0], 0.0)


def _conv2(x1, w, b):
    n = x1.shape[0]
    xq = x1.reshape(n, 6, 60, 2, 60, 2).transpose(0, 1, 3, 5, 2, 4)
    xq = jnp.pad(xq, ((0, 0), (0, 0), (0, 0), (0, 0), (0, 12), (0, 0)))
    w2 = w.reshape(16, 150)
    b2 = b.reshape(16, 1)
    return pl.pallas_call(
        _conv2_kernel,
        out_shape=jax.ShapeDtypeStruct((n, 16, 58, 58), jnp.float32),
        grid=(n,),
        in_specs=[
            pl.BlockSpec(memory_space=pltpu.SMEM),
            pl.BlockSpec(memory_space=pltpu.SMEM),
            pl.BlockSpec((1, 6, 2, 2, 72, 60),
                         lambda i: (i, 0, 0, 0, 0, 0)),
        ],
        out_specs=pl.BlockSpec((1, 16, 58, 58), lambda i: (i, 0, 0, 0)),
        scratch_shapes=[pltpu.VMEM((6, 2, 5, 72, 116), jnp.float32),
                        pltpu.VMEM((32, 64, 116), jnp.float32)],
        compiler_params=pltpu.CompilerParams(
            dimension_semantics=("parallel",)),
    )(w2, b2, xq)


FC1_TK = 8192


def _fc1_kernel(x_ref, w_ref, b_ref, o_ref, acc_ref):
    k = pl.program_id(1)

    @pl.when(k == 0)
    def _():
        acc_ref[...] = jnp.zeros_like(acc_ref)

    acc_ref[...] += jnp.dot(x_ref[...], w_ref[...],
                            preferred_element_type=jnp.float32)

    @pl.when(k == pl.num_programs(1) - 1)
    def _():
        o_ref[...] = jnp.maximum(acc_ref[...] + b_ref[...], 0.0)


def _fc23_kernel(h_ref, w2_ref, b2_ref, w3_ref, b3_ref, o_ref):
    h2 = jnp.maximum(
        jnp.dot(h_ref[...], w2_ref[...], preferred_element_type=jnp.float32)
        + b2_ref[...], 0.0)
    o_ref[...] = (jnp.dot(h2, w3_ref[...], preferred_element_type=jnp.float32)
                  + b3_ref[...])


def _fc_head(x2, w1, b1, w2, b2, w3, b3):
    m = x2.shape[0]
    kp = w1.shape[0]
    xp = jnp.pad(x2, ((0, 0), (0, kp - x2.shape[1])))
    nk = kp // FC1_TK
    h = pl.pallas_call(
        _fc1_kernel,
        out_shape=jax.ShapeDtypeStruct((m, 256), jnp.float32),
        grid=(2, nk),
        in_specs=[
            pl.BlockSpec((m, FC1_TK), lambda nh, k: (0, k)),
            pl.BlockSpec((FC1_TK, 128), lambda nh, k: (k, nh)),
            pl.BlockSpec((1, 128), lambda nh, k: (0, nh)),
        ],
        out_specs=pl.BlockSpec((m, 128), lambda nh, k: (0, nh)),
        scratch_shapes=[pltpu.VMEM((m, 128), jnp.float32)],
        compiler_params=pltpu.CompilerParams(
            dimension_semantics=("parallel", "arbitrary")),
    )(xp, w1, b1.reshape(1, 256))

    b2p = jnp.pad(b2, (0, 128 - b2.shape[0])).reshape(1, 128)
    b3p = jnp.pad(b3, (0, 128 - b3.shape[0])).reshape(1, 128)
    out = pl.pallas_call(
        _fc23_kernel,
        out_shape=jax.ShapeDtypeStruct((m, 128), jnp.float32),
    )(h, w2, b2p, w3, b3p)
    return out[:, :10]


def kernel(conv1_w, conv1_b, conv2_w, conv2_b, fc1_w_t, fc1_b,
           fc2_w_t, fc2_b, fc3_w_t, fc3_b, x):
    x1 = _conv1(x, conv1_w, conv1_b)
    x2 = _conv2(x1, conv2_w, conv2_b)
    n = x2.shape[0]
    flat = x2.reshape(n, 16 * 58 * 58)
    return _fc_head(flat, fc1_w_t, fc1_b, fc2_w_t, fc2_b, fc3_w_t, fc3_b)


# speedup vs baseline: 20.1649x; 1.2184x over previous
import jax
import jax.numpy as jnp
from jax import lax
from jax.experimental import pallas as pl
from jax.experimental.pallas import tpu as pltpu


def _conv1_kernel(w_ref, b_ref, xr_ref, p_ref, o_ref, xs_ref, acc_ref):
    n_co = 6
    phases = ((0, 0), (0, 1), (1, 0), (1, 1))

    for ci in range(3):
        for by in range(2):
            for s in range(6):
                xs_ref[ci, by, s] = jnp.dot(
                    xr_ref[0, ci, by], p_ref[s],
                    preferred_element_type=jnp.float32)

    acc_ref[...] = jnp.zeros_like(acc_ref)

    def tap_body(cidy, carry):
        ci = cidy // 5
        dy = cidy % 5
        ws = [[w_ref[co, cidy * 5 + dx] for dx in range(5)]
              for co in range(n_co)]
        bys = []
        ays = []
        for ry in range(2):
            bys.append((ry + dy) % 2)
            ays.append((ry + dy) // 2)
        for st in range(5):
            p0 = 24 * st
            sl = [[xs_ref[ci, bys[ry], s, pl.ds(p0 + ays[ry], 24), :]
                   for s in range(6)] for ry in range(2)]
            for ph, (ry, rx) in enumerate(phases):
                for co in range(n_co):
                    a = acc_ref[ph * n_co + co, pl.ds(p0, 24), :]
                    for dx in range(5):
                        a = a + ws[co][dx] * sl[ry][rx + dx]
                    acc_ref[ph * n_co + co, pl.ds(p0, 24), :] = a
        return carry

    lax.fori_loop(0, 15, tap_body, 0)

    for co in range(n_co):
        m = jnp.maximum(jnp.maximum(acc_ref[co], acc_ref[n_co + co]),
                        jnp.maximum(acc_ref[2 * n_co + co],
                                    acc_ref[3 * n_co + co]))
        o_ref[0, co] = jnp.maximum(m[:, :120] + b_ref[co, 0], 0.0)


def _conv1(x, w, b):
    n = x.shape[0]
    xr = x.reshape(n, 3, 122, 2, 244).transpose(0, 1, 3, 2, 4)
    c = jnp.arange(128)
    s = jnp.arange(6)
    src = 2 * c[None, :] + s[:, None]
    p1 = ((jnp.arange(244)[None, :, None] == src[:, None, :]) &
          (c[None, None, :] < 120)).astype(jnp.float32)
    w2 = w.reshape(6, 75)
    b2 = b.reshape(6, 1)
    return pl.pallas_call(
        _conv1_kernel,
        out_shape=jax.ShapeDtypeStruct((n, 6, 120, 120), jnp.float32),
        grid=(n,),
        in_specs=[
            pl.BlockSpec(memory_space=pltpu.SMEM),
            pl.BlockSpec(memory_space=pltpu.SMEM),
            pl.BlockSpec((1, 3, 2, 122, 244), lambda i: (i, 0, 0, 0, 0)),
            pl.BlockSpec((6, 244, 128), lambda i: (0, 0, 0)),
        ],
        out_specs=pl.BlockSpec((1, 6, 120, 120), lambda i: (i, 0, 0, 0)),
        scratch_shapes=[pltpu.VMEM((3, 2, 6, 122, 128), jnp.float32),
                        pltpu.VMEM((24, 120, 128), jnp.float32)],
        compiler_params=pltpu.CompilerParams(
            dimension_semantics=("parallel",)),
    )(w2, b2, xr, p1)


def _conv2_kernel(w_ref, b_ref, xr_ref, p_ref, o_ref, xp_ref, acc_ref):
    n_co = 16

    for ci in range(6):
        for by in range(2):
            for dx in range(5):
                xp_ref[ci, by, dx] = jnp.dot(
                    xr_ref[0, ci, by], p_ref[dx],
                    preferred_element_type=jnp.float32)

    acc_ref[...] = jnp.zeros_like(acc_ref)

    def tap_body(cidy, carry):
        ci = cidy // 5
        dy = cidy % 5
        ws = [[w_ref[co, cidy * 5 + dx] for dx in range(5)]
              for co in range(n_co)]
        for st in range(4):
            p0 = 16 * st
            sl = []
            for ry in range(2):
                by = (ry + dy) % 2
                ay = (ry + dy) // 2
                sl.append([xp_ref[ci, by, dx, pl.ds(p0 + ay, 16), :]
                           for dx in range(5)])
            for ry in range(2):
                for co in range(n_co):
                    a = acc_ref[ry * n_co + co, pl.ds(p0, 16), :]
                    for dx in range(5):
                        a = a + ws[co][dx] * sl[ry][dx]
                    acc_ref[ry * n_co + co, pl.ds(p0, 16), :] = a
        return carry

    lax.fori_loop(0, 30, tap_body, 0)

    for co in range(n_co):
        m = jnp.maximum(acc_ref[co], acc_ref[n_co + co])
        m = jnp.maximum(m[:, :58], m[:, 58:116])
        o_ref[0, co] = jnp.maximum(m[:58, :] + b_ref[co, 0], 0.0)


def _conv2(x1, w, b):
    n = x1.shape[0]
    xr = x1.reshape(n, 6, 60, 2, 120).transpose(0, 1, 3, 2, 4)
    xr = jnp.pad(xr, ((0, 0), (0, 0), (0, 0), (0, 12), (0, 0)))
    c = jnp.arange(128)
    dxb = jnp.arange(5)
    src0 = 2 * c[None, :] + dxb[:, None]
    src1 = 2 * (c[None, :] - 58) + 1 + dxb[:, None]
    src = jnp.where(c[None, :] < 58, src0, src1)
    p2 = ((jnp.arange(120)[None, :, None] == src[:, None, :]) &
          (c[None, None, :] < 116)).astype(jnp.float32)
    w2 = w.reshape(16, 150)
    b2 = b.reshape(16, 1)
    return pl.pallas_call(
        _conv2_kernel,
        out_shape=jax.ShapeDtypeStruct((n, 16, 58, 58), jnp.float32),
        grid=(n,),
        in_specs=[
            pl.BlockSpec(memory_space=pltpu.SMEM),
            pl.BlockSpec(memory_space=pltpu.SMEM),
            pl.BlockSpec((1, 6, 2, 72, 120), lambda i: (i, 0, 0, 0, 0)),
            pl.BlockSpec((5, 120, 128), lambda i: (0, 0, 0)),
        ],
        out_specs=pl.BlockSpec((1, 16, 58, 58), lambda i: (i, 0, 0, 0)),
        scratch_shapes=[pltpu.VMEM((6, 2, 5, 72, 128), jnp.float32),
                        pltpu.VMEM((32, 64, 128), jnp.float32)],
        compiler_params=pltpu.CompilerParams(
            dimension_semantics=("parallel",)),
    )(w2, b2, xr, p2)


FC1_TK = 8192


def _fc1_kernel(x_ref, w_ref, b_ref, o_ref, acc_ref):
    k = pl.program_id(1)

    @pl.when(k == 0)
    def _():
        acc_ref[...] = jnp.zeros_like(acc_ref)

    acc_ref[...] += jnp.dot(x_ref[...], w_ref[...],
                            preferred_element_type=jnp.float32)

    @pl.when(k == pl.num_programs(1) - 1)
    def _():
        o_ref[...] = jnp.maximum(acc_ref[...] + b_ref[...], 0.0)


def _fc23_kernel(h_ref, w2_ref, b2_ref, w3_ref, b3_ref, o_ref):
    h2 = jnp.maximum(
        jnp.dot(h_ref[...], w2_ref[...], preferred_element_type=jnp.float32)
        + b2_ref[...], 0.0)
    o_ref[...] = (jnp.dot(h2, w3_ref[...], preferred_element_type=jnp.float32)
                  + b3_ref[...])


def _fc_head(x2, w1, b1, w2, b2, w3, b3):
    m = x2.shape[0]
    kp = w1.shape[0]
    xp = jnp.pad(x2, ((0, 0), (0, kp - x2.shape[1])))
    nk = kp // FC1_TK
    h = pl.pallas_call(
        _fc1_kernel,
        out_shape=jax.ShapeDtypeStruct((m, 256), jnp.float32),
        grid=(2, nk),
        in_specs=[
            pl.BlockSpec((m, FC1_TK), lambda nh, k: (0, k)),
            pl.BlockSpec((FC1_TK, 128), lambda nh, k: (k, nh)),
            pl.BlockSpec((1, 128), lambda nh, k: (0, nh)),
        ],
        out_specs=pl.BlockSpec((m, 128), lambda nh, k: (0, nh)),
        scratch_shapes=[pltpu.VMEM((m, 128), jnp.float32)],
        compiler_params=pltpu.CompilerParams(
            dimension_semantics=("parallel", "arbitrary")),
    )(xp, w1, b1.reshape(1, 256))

    b2p = jnp.pad(b2, (0, 128 - b2.shape[0])).reshape(1, 128)
    b3p = jnp.pad(b3, (0, 128 - b3.shape[0])).reshape(1, 128)
    out = pl.pallas_call(
        _fc23_kernel,
        out_shape=jax.ShapeDtypeStruct((m, 128), jnp.float32),
    )(h, w2, b2p, w3, b3p)
    return out[:, :10]


def kernel(conv1_w, conv1_b, conv2_w, conv2_b, fc1_w_t, fc1_b,
           fc2_w_t, fc2_b, fc3_w_t, fc3_b, x):
    x1 = _conv1(x, conv1_w, conv1_b)
    x2 = _conv2(x1, conv2_w, conv2_b)
    n = x2.shape[0]
    flat = x2.reshape(n, 16 * 58 * 58)
    return _fc_head(flat, fc1_w_t, fc1_b, fc2_w_t, fc2_b, fc3_w_t, fc3_b)
